# baseline (speedup 1.0000x reference)
import functools
import math

import jax
import jax.numpy as jnp
from jax import lax
from jax.experimental import pallas as pl
from jax.experimental.pallas import tpu as pltpu
from jax.experimental.pallas import tpu_sc as plsc

_N = 200000
_CHUNK = 1024
_NFULL = _N // _CHUNK
_TAIL = _N - _NFULL * _CHUNK
_NCHUNKS = _NFULL + 1
_NW = 32
_MAX_CHUNKS_PER_W = -(-_NCHUNKS // _NW)
_NSEG = 16000
_ACC = 64128
_PAD = -999
_TEMPERATURE = 298.0
_DCP = (0.008 - 5e-05 * (_TEMPERATURE - 273.0)) * math.log(_TEMPERATURE / 273)
_CORR_IONIC = math.sqrt(0.05) / 3.9

_mesh = plsc.VectorSubcoreMesh(core_axis_name="c", subcore_axis_name="s",
                               num_cores=1, num_subcores=16)
_CMID = 98


def _make_solv(c_lo, c_hi):
  maxk = -(-(c_hi - c_lo) // 16)

  @functools.partial(
    pl.kernel,
    out_type=(jax.ShapeDtypeStruct((_ACC,), jnp.float32),
              jax.ShapeDtypeStruct((_ACC,), jnp.float32)),
    mesh=_mesh,
    compiler_params=pltpu.CompilerParams(needs_layout_passes=False),
    scratch_types=(
        pltpu.VMEM((1024,), jnp.int32),
        pltpu.VMEM((1024,), jnp.int32),
        pltpu.VMEM((1024,), jnp.int32),
        pltpu.VMEM((1024,), jnp.int32),
        pltpu.VMEM((4096,), jnp.float32),
        pltpu.VMEM((4096,), jnp.float32),
        pltpu.VMEM((4096,), jnp.int32),
        pltpu.VMEM((16384,), jnp.float32),
        pltpu.VMEM((512,), jnp.float32),
        pltpu.VMEM((512,), jnp.int32),
        pltpu.VMEM((3, 16), jnp.float32),
        pltpu.VMEM((32, 128), jnp.int32),
        pltpu.VMEM((32, 128), jnp.float32),
        pltpu.VMEM((32, 128), jnp.float32),
        pltpu.VMEM((4008,), jnp.float32),
        pltpu.VMEM_SHARED((_ACC,), jnp.float32),
        pltpu.VMEM_SHARED((_ACC,), jnp.float32),
        pltpu.SemaphoreType.DMA,
    ),
  )
  def _sc_solv(desc_h, facc_h, cr_h, hb_h, crp_h, props_h,
             frot_h, consts_h, zf_h, zf2_h, zi2_h,
             desc_tl, facc_tl, cr_tl, hb_tl, crp_tl, outP, outH,
             bb_v, cc_v, rr_v, tt_v, facc_v, cr_v, hb_v, crp_v,
             props_v, frot_v, consts_v, segel_v, valP_v, valH_v, zbuf_v,
             accP, accH, dsem):
    sidx = lax.axis_index("s")
    wid = sidx

    pltpu.sync_copy(props_h, props_v.at[pl.ds(0, 500)])
    pltpu.sync_copy(frot_h, frot_v.at[pl.ds(0, 400)])
    pltpu.sync_copy(consts_h, consts_v)
    pltpu.sync_copy(zf2_h, valP_v)
    pltpu.sync_copy(zf2_h, valH_v)
    pltpu.sync_copy(zi2_h, segel_v)
    pltpu.sync_copy(zf_h.at[pl.ds(0, 4008)], zbuf_v)
    zbase = sidx * 4008
    pltpu.sync_copy(zbuf_v, accP.at[pl.ds(zbase, 4008)])
    pltpu.sync_copy(zbuf_v, accH.at[pl.ds(zbase, 4008)])

    plsc.subcore_barrier()

    lane = lax.iota(jnp.int32, 16)
    cH_v = consts_v[0]
    cPW_v = consts_v[1]
    cP_v = consts_v[2]
    f0 = jnp.float32(0.0)
    f1 = jnp.float32(1.0)
    f2 = jnp.float32(2.0)

    def load_chunk(a0, segs):
        cps = []
        for off, sz in segs:
            for dst, i in ((bb_v, 0), (cc_v, 1), (rr_v, 2), (tt_v, 3)):
                cps.append(pltpu.async_copy(desc_h.at[i, pl.ds(a0 + off, sz)],
                                            dst.at[pl.ds(off, sz)], dsem))
            for hsrc, dst in ((facc_h, facc_v), (cr_h, cr_v), (hb_h, hb_v)):
                for j in range(4):
                    cps.append(pltpu.async_copy(
                        hsrc.at[j, pl.ds(a0 + off, sz)],
                        dst.at[pl.ds(j * 1024 + off, sz)], dsem))
            for j in range(4):
                for kk in range(4):
                    cps.append(pltpu.async_copy(
                        crp_h.at[j, kk, pl.ds(a0 + off, sz)],
                        crp_v.at[pl.ds((j * 4 + kk) * 1024 + off, sz)], dsem))
        for cp in cps:
            cp.wait()

    def load_tail64():
        cps = []
        for dst, i in ((bb_v, 0), (cc_v, 1), (rr_v, 2), (tt_v, 3)):
            cps.append(pltpu.async_copy(desc_tl.at[pl.ds(i * 64, 64)],
                                        dst.at[pl.ds(256, 64)], dsem))
        for hsrc, dst in ((facc_tl, facc_v), (cr_tl, cr_v), (hb_tl, hb_v)):
            for j in range(4):
                cps.append(pltpu.async_copy(
                    hsrc.at[pl.ds(j * 64, 64)],
                    dst.at[pl.ds(j * 1024 + 256, 64)], dsem))
        for j in range(4):
            for kk in range(4):
                cps.append(pltpu.async_copy(
                    crp_tl.at[pl.ds((j * 4 + kk) * 64, 64)],
                    crp_v.at[pl.ds((j * 4 + kk) * 1024 + 256, 64)], dsem))
        for cp in cps:
            cp.wait()

    def group_body(g):
        o = g * 16
        bv = bb_v[pl.ds(o, 16)]
        cv = cc_v[pl.ds(o, 16)]
        rv = rr_v[pl.ds(o, 16)]
        tv = tt_v[pl.ds(o, 16)]
        valid = rv != _PAD
        segv = bv * 8000 + cv * 2000 + jnp.where(valid, rv, 0)
        segv = jnp.where(valid, segv, _NSEG)
        seg4 = jnp.clip(segv, 0, _NSEG) * 4
        enerG = plsc.load_gather(props_v, [tv])
        virt = plsc.load_gather(props_v, [tv + 100])
        chg = plsc.load_gather(props_v, [tv + 200])
        occ = plsc.load_gather(props_v, [tv + 300])
        occm = plsc.load_gather(props_v, [tv + 400])
        fr = [plsc.load_gather(frot_v, [tv + 100 * kk]) for kk in range(4)]
        is_nit = (tv == 10) | (tv == 11)
        flat_acc = (tv >= 5) & (tv <= 8)
        chg_ok = chg != jnp.float32(_PAD)
        lin = chg_ok & (~is_nit) & flat_acc
        sfac = jnp.where(lin, f1, f2)
        ex = [f != _PAD for f in fr]
        right = [[f == kk for f in fr] for kk in range(2)]
        anyr = [(r[0] | r[1]) | (r[2] | r[3]) for r in right]
        zden_chg = (occm != occ) & chg_ok
        denom = occm - occ
        denom_safe = jnp.where(denom == f0, f1, denom)
        inv_den = f1 / denom_safe
        padding = virt == f0
        hydro = (enerG <= f0) & padding
        nonh = (enerG >= f0) & padding
        hterm = cH_v * (enerG - jnp.float32(_DCP))
        for j in range(4):
            oj = j * 1024 + o
            faccl = facc_v[pl.ds(oj, 16)]
            crl = cr_v[pl.ds(oj, 16)]
            hbl = hb_v[pl.ds(oj, 16)]
            ok = (j * 4) * 1024 + o
            crp = [crp_v[pl.ds(ok + kk * 1024, 16)] for kk in range(4)]
            gt0 = [x > f0 for x in crp]
            s = ((crp[0] + crp[1]) + (crp[2] + crp[3])) * sfac
            ucr = s + crl
            all_fake = None
            for kk in range(2):
                tmp = [(~right[kk][i]) | gt0[i] for i in range(4)]
                allt = (tmp[0] & tmp[1]) & (tmp[2] & tmp[3])
                af_k = allt & anyr[kk]
                all_fake = af_k if all_fake is None else (all_fake | af_k)
            eaz = ((ex[0] & (~gt0[0])) & (ex[1] & (~gt0[1]))
                   & (ex[2] & (~gt0[2])) & (ex[3] & (~gt0[3])))
            spcm = (s > f0) & zden_chg & all_fake & (hbl == 0) & (~eaz)
            val = jnp.clip((ucr - occ) * inv_den, jnp.float32(1e-10), f2)
            faccPol = jnp.where(spcm, val, f0)
            solvH = jnp.where(hydro, hterm * faccl, f0)
            solvP = jnp.where(nonh & spcm, cPW_v * faccPol,
                              jnp.where(nonh & (~spcm), cP_v * faccl,
                                        f0)) * enerG
            e = (o + lane) * 4 + j
            erow = jnp.right_shift(e, 7)
            ecol = jnp.bitwise_and(e, 127)
            plsc.store_scatter(segel_v, [erow, ecol], seg4 + j)
            plsc.store_scatter(valP_v, [erow, ecol], solvP)
            plsc.store_scatter(valH_v, [erow, ecol], solvH)

    @pl.loop(0, maxk)
    def _chunk(k):
        cid = c_lo + wid + k * 16

        @pl.when((cid < c_hi) & (cid < _NFULL))
        def _():
            load_chunk(cid * _CHUNK, [(0, _CHUNK)])

            @pl.loop(0, _CHUNK // 16)
            def _g(g):
                group_body(g)

        @pl.when((cid == _NFULL) & (cid < c_hi))
        def _():
            load_chunk(_NFULL * _CHUNK, [(0, 256)])
            load_tail64()
            pltpu.sync_copy(zf2_h.at[pl.ds(8, 24)], valP_v.at[pl.ds(8, 24)])
            pltpu.sync_copy(zf2_h.at[pl.ds(8, 24)], valH_v.at[pl.ds(8, 24)])

            @pl.loop(0, _TAIL // 16)
            def _g(g):
                group_body(g)

        @pl.when(cid < c_hi)
        def _():
            @pl.loop(0, 4)
            def _scat(jj):
                j0 = jj * 8
                sps = []
                for i in range(8):
                    sps.append(pltpu.async_copy(
                        valP_v.at[j0 + i], accP.at[segel_v.at[j0 + i]],
                        dsem, add=True))
                    sps.append(pltpu.async_copy(
                        valH_v.at[j0 + i], accH.at[segel_v.at[j0 + i]],
                        dsem, add=True))
                for sp in sps:
                    sp.wait()

    plsc.subcore_barrier()
    ebase = sidx * 4008
    pltpu.sync_copy(accP.at[pl.ds(ebase, 4008)], facc_v.at[pl.ds(0, 4008)])
    pltpu.sync_copy(facc_v.at[pl.ds(0, 4008)], outP.at[pl.ds(ebase, 4008)])
    pltpu.sync_copy(accH.at[pl.ds(ebase, 4008)], cr_v.at[pl.ds(0, 4008)])
    pltpu.sync_copy(cr_v.at[pl.ds(0, 4008)], outH.at[pl.ds(ebase, 4008)])

  return _sc_solv


_solv_a = _make_solv(0, _CMID)
_solv_b = _make_solv(_CMID, _NCHUNKS)


def _tc_add(pa_ref, pb_ref, ha_ref, hb_ref, oP_ref, oH_ref):
    oP_ref[...] = pa_ref[:500] + pb_ref[:500]
    oH_ref[...] = ha_ref[:500] + hb_ref[:500]


def kernel(atom_description, facc, contRat, contRatPol, atomHbonds, atomDisul,
           atom_Properties, fake_rot, w_hydro, w_polar, w_polar_water):
    del atomDisul
    desc_t = atom_description.astype(jnp.int32).T
    facc_t = facc.T
    cr_t = contRat.T
    hb_t = atomHbonds.astype(jnp.int32).T
    crp_t = jnp.transpose(contRatPol, (1, 2, 0))
    props_f = atom_Properties.T.reshape(-1)
    frot_f = fake_rot.astype(jnp.int32).T.reshape(-1)
    cH = (1.0 - jnp.tanh(-w_hydro[0])) * (1.0 + _CORR_IONIC)
    cPW = 1.0 - jnp.tanh(-w_polar_water[0])
    cP = 1.0 - jnp.tanh(-w_polar[0])
    consts = jnp.broadcast_to(
        jnp.stack([cH, cPW, cP]).astype(jnp.float32)[:, None], (3, 16))
    zf = jnp.zeros((_ACC,), jnp.float32)
    zf2 = jnp.zeros((32, 128), jnp.float32)
    zi2 = jnp.zeros((32, 128), jnp.int32)
    t0 = _N - 64
    desc_tl = atom_description[t0:].astype(jnp.int32).T.reshape(-1)
    facc_tl = facc[t0:].T.reshape(-1)
    cr_tl = contRat[t0:].T.reshape(-1)
    hb_tl = atomHbonds[t0:].astype(jnp.int32).T.reshape(-1)
    crp_tl = jnp.transpose(contRatPol[t0:], (1, 2, 0)).reshape(-1)

    args = (desc_t, facc_t, cr_t, hb_t, crp_t,
            props_f, frot_f, consts, zf, zf2, zi2,
            desc_tl, facc_tl, cr_tl, hb_tl, crp_tl)
    Pa, Ha = _solv_a(*args)
    Pb, Hb = _solv_b(*args)

    P, H = pl.pallas_call(
        _tc_add,
        out_shape=(jax.ShapeDtypeStruct((500, 128), jnp.float32),
                   jax.ShapeDtypeStruct((500, 128), jnp.float32)),
    )(Pa.reshape(501, 128), Pb.reshape(501, 128),
      Ha.reshape(501, 128), Hb.reshape(501, 128))
    return P.reshape(2, 4, 2000, 4), H.reshape(2, 4, 2000, 4)


# speedup vs baseline: 1.5728x; 1.5728x over previous
import functools
import math

import jax
import jax.numpy as jnp
from jax import lax
from jax.experimental import pallas as pl
from jax.experimental.pallas import tpu as pltpu
from jax.experimental.pallas import tpu_sc as plsc

_N = 200000
_CHUNK = 1024
_NFULL = _N // _CHUNK
_TAIL = _N - _NFULL * _CHUNK
_NCHUNKS = _NFULL + 1
_NW = 32
_MAXK = -(-_NCHUNKS // _NW)
_NSEG = 16000
_ACC = 64128
_PAD = -999
_TEMPERATURE = 298.0
_DCP = (0.008 - 5e-05 * (_TEMPERATURE - 273.0)) * math.log(_TEMPERATURE / 273)
_CORR_IONIC = math.sqrt(0.05) / 3.9

_mesh = plsc.VectorSubcoreMesh(core_axis_name="c", subcore_axis_name="s",
                               num_cores=2, num_subcores=16)


@functools.partial(
    pl.kernel,
    out_type=(jax.ShapeDtypeStruct((2 * _ACC,), jnp.float32),
              jax.ShapeDtypeStruct((2 * _ACC,), jnp.float32)),
    mesh=_mesh,
    compiler_params=pltpu.CompilerParams(needs_layout_passes=False),
    scratch_types=(
        [[pltpu.VMEM((1024,), jnp.int32)] * 4
         + [pltpu.VMEM((4096,), jnp.float32),
            pltpu.VMEM((4096,), jnp.float32),
            pltpu.VMEM((4096,), jnp.int32),
            pltpu.VMEM((16384,), jnp.float32)]
         for _ in range(2)],
        [[pltpu.VMEM((32, 128), jnp.int32),
          pltpu.VMEM((32, 128), jnp.float32),
          pltpu.VMEM((32, 128), jnp.float32)]
         for _ in range(2)],
        pltpu.VMEM((512,), jnp.float32),
        pltpu.VMEM((512,), jnp.int32),
        pltpu.VMEM((3, 16), jnp.float32),
        pltpu.VMEM((4008,), jnp.float32),
        pltpu.VMEM_SHARED((_ACC,), jnp.float32),
        pltpu.VMEM_SHARED((_ACC,), jnp.float32),
        [pltpu.SemaphoreType.DMA] * 2,
        [pltpu.SemaphoreType.DMA] * 2,
    ),
)
def _sc_solv(desc_h, facc_h, cr_h, hb_h, crp_h, props_h,
             frot_h, consts_h, zf_h, zf2_h, zi2_h,
             desc_tl, facc_tl, cr_tl, hb_tl, crp_tl, outP, outH,
             inbuf, stage, props_v, frot_v, consts_v, zbuf_v,
             accP, accH, isem, ssem):
    cidx = lax.axis_index("c")
    sidx = lax.axis_index("s")
    wid = cidx * 16 + sidx

    pltpu.sync_copy(props_h, props_v.at[pl.ds(0, 500)])
    pltpu.sync_copy(frot_h, frot_v.at[pl.ds(0, 400)])
    pltpu.sync_copy(consts_h, consts_v)
    for b in range(2):
        pltpu.sync_copy(zi2_h, stage[b][0])
        pltpu.sync_copy(zf2_h, stage[b][1])
        pltpu.sync_copy(zf2_h, stage[b][2])
    pltpu.sync_copy(zf_h.at[pl.ds(0, 4008)], zbuf_v)
    zbase = sidx * 4008
    pltpu.sync_copy(zbuf_v, accP.at[pl.ds(zbase, 4008)])
    pltpu.sync_copy(zbuf_v, accH.at[pl.ds(zbase, 4008)])

    plsc.subcore_barrier()

    lane = lax.iota(jnp.int32, 16)
    cH_v = consts_v[0]
    cPW_v = consts_v[1]
    cP_v = consts_v[2]
    f0 = jnp.float32(0.0)
    f1 = jnp.float32(1.0)
    f2 = jnp.float32(2.0)

    def in_descs(b, a0, segs, make_only):
        bb_v, cc_v, rr_v, tt_v, facc_v, cr_v, hb_v, crp_v = inbuf[b]
        api = pltpu.make_async_copy if make_only else pltpu.async_copy
        cps = []
        for off, sz in segs:
            for dst, i in ((bb_v, 0), (cc_v, 1), (rr_v, 2), (tt_v, 3)):
                cps.append(api(desc_h.at[i, pl.ds(a0 + off, sz)],
                               dst.at[pl.ds(off, sz)], isem[b]))
            for hsrc, dst in ((facc_h, facc_v), (cr_h, cr_v), (hb_h, hb_v)):
                for j in range(4):
                    cps.append(api(hsrc.at[j, pl.ds(a0 + off, sz)],
                                   dst.at[pl.ds(j * 1024 + off, sz)],
                                   isem[b]))
            for j in range(4):
                for kk in range(4):
                    cps.append(api(
                        crp_h.at[j, kk, pl.ds(a0 + off, sz)],
                        crp_v.at[pl.ds((j * 4 + kk) * 1024 + off, sz)],
                        isem[b]))
        return cps

    def tail64_descs(b, make_only):
        bb_v, cc_v, rr_v, tt_v, facc_v, cr_v, hb_v, crp_v = inbuf[b]
        api = pltpu.make_async_copy if make_only else pltpu.async_copy
        cps = []
        for dst, i in ((bb_v, 0), (cc_v, 1), (rr_v, 2), (tt_v, 3)):
            cps.append(api(desc_tl.at[pl.ds(i * 64, 64)],
                           dst.at[pl.ds(256, 64)], isem[b]))
        for hsrc, dst in ((facc_tl, facc_v), (cr_tl, cr_v), (hb_tl, hb_v)):
            for j in range(4):
                cps.append(api(hsrc.at[pl.ds(j * 64, 64)],
                               dst.at[pl.ds(j * 1024 + 256, 64)], isem[b]))
        for j in range(4):
            for kk in range(4):
                cps.append(api(crp_tl.at[pl.ds((j * 4 + kk) * 64, 64)],
                               crp_v.at[pl.ds((j * 4 + kk) * 1024 + 256, 64)],
                               isem[b]))
        return cps

    def fire_full(b, cid):
        in_descs(b, cid * _CHUNK, [(0, _CHUNK)], False)

    def fire_tail(b):
        in_descs(b, _NFULL * _CHUNK, [(0, 256)], False)
        tail64_descs(b, False)

    def drain_full(b):
        for cp in in_descs(b, 0, [(0, _CHUNK)], True):
            cp.wait()

    def drain_tail(b):
        for cp in in_descs(b, _NFULL * _CHUNK, [(0, 256)], True):
            cp.wait()
        for cp in tail64_descs(b, True):
            cp.wait()

    def group_body(b, g):
        bb_v, cc_v, rr_v, tt_v, facc_v, cr_v, hb_v, crp_v = inbuf[b]
        segel_v, valP_v, valH_v = stage[b]
        o = g * 16
        bv = bb_v[pl.ds(o, 16)]
        cv = cc_v[pl.ds(o, 16)]
        rv = rr_v[pl.ds(o, 16)]
        tv = tt_v[pl.ds(o, 16)]
        valid = rv != _PAD
        segv = bv * 8000 + cv * 2000 + jnp.where(valid, rv, 0)
        segv = jnp.where(valid, segv, _NSEG)
        seg4 = jnp.clip(segv, 0, _NSEG) * 4
        enerG = plsc.load_gather(props_v, [tv])
        virt = plsc.load_gather(props_v, [tv + 100])
        chg = plsc.load_gather(props_v, [tv + 200])
        occ = plsc.load_gather(props_v, [tv + 300])
        occm = plsc.load_gather(props_v, [tv + 400])
        fr = [plsc.load_gather(frot_v, [tv + 100 * kk]) for kk in range(4)]
        is_nit = (tv == 10) | (tv == 11)
        flat_acc = (tv >= 5) & (tv <= 8)
        chg_ok = chg != jnp.float32(_PAD)
        lin = chg_ok & (~is_nit) & flat_acc
        sfac = jnp.where(lin, f1, f2)
        ex = [f != _PAD for f in fr]
        right = [[f == kk for f in fr] for kk in range(2)]
        anyr = [(r[0] | r[1]) | (r[2] | r[3]) for r in right]
        zden_chg = (occm != occ) & chg_ok
        denom = occm - occ
        denom_safe = jnp.where(denom == f0, f1, denom)
        inv_den = f1 / denom_safe
        padding = virt == f0
        hydro = (enerG <= f0) & padding
        nonh = (enerG >= f0) & padding
        hterm = cH_v * (enerG - jnp.float32(_DCP))
        for j in range(4):
            oj = j * 1024 + o
            faccl = facc_v[pl.ds(oj, 16)]
            crl = cr_v[pl.ds(oj, 16)]
            hbl = hb_v[pl.ds(oj, 16)]
            ok = (j * 4) * 1024 + o
            crp = [crp_v[pl.ds(ok + kk * 1024, 16)] for kk in range(4)]
            gt0 = [x > f0 for x in crp]
            s = ((crp[0] + crp[1]) + (crp[2] + crp[3])) * sfac
            ucr = s + crl
            all_fake = None
            for kk in range(2):
                tmp = [(~right[kk][i]) | gt0[i] for i in range(4)]
                allt = (tmp[0] & tmp[1]) & (tmp[2] & tmp[3])
                af_k = allt & anyr[kk]
                all_fake = af_k if all_fake is None else (all_fake | af_k)
            eaz = ((ex[0] & (~gt0[0])) & (ex[1] & (~gt0[1]))
                   & (ex[2] & (~gt0[2])) & (ex[3] & (~gt0[3])))
            spcm = (s > f0) & zden_chg & all_fake & (hbl == 0) & (~eaz)
            val = jnp.clip((ucr - occ) * inv_den, jnp.float32(1e-10), f2)
            faccPol = jnp.where(spcm, val, f0)
            solvH = jnp.where(hydro, hterm * faccl, f0)
            solvP = jnp.where(nonh & spcm, cPW_v * faccPol,
                              jnp.where(nonh & (~spcm), cP_v * faccl,
                                        f0)) * enerG
            e = (o + lane) * 4 + j
            erow = jnp.right_shift(e, 7)
            ecol = jnp.bitwise_and(e, 127)
            plsc.store_scatter(segel_v, [erow, ecol], seg4 + j)
            plsc.store_scatter(valP_v, [erow, ecol], solvP)
            plsc.store_scatter(valH_v, [erow, ecol], solvH)

    def fire_scatter(b):
        segel_v, valP_v, valH_v = stage[b]

        @pl.loop(0, 4)
        def _scat(jj):
            j0 = jj * 8
            for i in range(8):
                pltpu.async_copy(valP_v.at[j0 + i],
                                 accP.at[segel_v.at[j0 + i]],
                                 ssem[b], add=True)
                pltpu.async_copy(valH_v.at[j0 + i],
                                 accH.at[segel_v.at[j0 + i]],
                                 ssem[b], add=True)

    def drain_scatter(b):
        pltpu.make_async_copy(zf2_h, stage[b][1], ssem[b]).wait()
        pltpu.make_async_copy(zf2_h, stage[b][2], ssem[b]).wait()

    fire_full(0, wid)
    for k in range(_MAXK):
        b = k % 2
        cid = wid + k * _NW
        nxt = wid + (k + 1) * _NW
        if k + 1 < _MAXK:
            if k + 1 == _MAXK - 1:
                @pl.when(nxt < _NFULL)
                def _():
                    fire_full(1 - b, nxt)

                @pl.when(nxt == _NFULL)
                def _():
                    fire_tail(1 - b)
            else:
                fire_full(1 - b, nxt)
        if k >= 2:
            drain_scatter(b)
        if k < _MAXK - 1:
            drain_full(b)

            @pl.loop(0, _CHUNK // 16)
            def _g(g):
                group_body(b, g)

            fire_scatter(b)
        else:
            @pl.when(cid < _NFULL)
            def _():
                drain_full(b)

                @pl.loop(0, _CHUNK // 16)
                def _g(g):
                    group_body(b, g)

                fire_scatter(b)

            @pl.when(cid == _NFULL)
            def _():
                drain_tail(b)
                pltpu.sync_copy(zf2_h.at[pl.ds(8, 24)],
                                stage[b][1].at[pl.ds(8, 24)])
                pltpu.sync_copy(zf2_h.at[pl.ds(8, 24)],
                                stage[b][2].at[pl.ds(8, 24)])

                @pl.loop(0, _TAIL // 16)
                def _g(g):
                    group_body(b, g)

                fire_scatter(b)

    drain_scatter((_MAXK - 2) % 2)
    last = _MAXK - 1

    @pl.when(wid + last * _NW < _NCHUNKS)
    def _():
        drain_scatter(last % 2)

    plsc.subcore_barrier()
    ebase = sidx * 4000
    obase = cidx * _ACC + ebase
    tmpP = inbuf[0][4]
    tmpH = inbuf[0][5]
    pltpu.sync_copy(accP.at[pl.ds(ebase, 4000)], tmpP.at[pl.ds(0, 4000)])
    pltpu.sync_copy(tmpP.at[pl.ds(0, 4000)], outP.at[pl.ds(obase, 4000)])
    pltpu.sync_copy(accH.at[pl.ds(ebase, 4000)], tmpH.at[pl.ds(0, 4000)])
    pltpu.sync_copy(tmpH.at[pl.ds(0, 4000)], outH.at[pl.ds(obase, 4000)])


def _tc_add(p_ref, h_ref, oP_ref, oH_ref):
    oP_ref[...] = p_ref[0, :500] + p_ref[1, :500]
    oH_ref[...] = h_ref[0, :500] + h_ref[1, :500]


def kernel(atom_description, facc, contRat, contRatPol, atomHbonds, atomDisul,
           atom_Properties, fake_rot, w_hydro, w_polar, w_polar_water):
    del atomDisul
    desc_t = atom_description.astype(jnp.int32).T
    facc_t = facc.T
    cr_t = contRat.T
    hb_t = atomHbonds.astype(jnp.int32).T
    crp_t = jnp.transpose(contRatPol, (1, 2, 0))
    props_f = atom_Properties.T.reshape(-1)
    frot_f = fake_rot.astype(jnp.int32).T.reshape(-1)
    cH = (1.0 - jnp.tanh(-w_hydro[0])) * (1.0 + _CORR_IONIC)
    cPW = 1.0 - jnp.tanh(-w_polar_water[0])
    cP = 1.0 - jnp.tanh(-w_polar[0])
    consts = jnp.broadcast_to(
        jnp.stack([cH, cPW, cP]).astype(jnp.float32)[:, None], (3, 16))
    zf = jnp.zeros((_ACC,), jnp.float32)
    zf2 = jnp.zeros((32, 128), jnp.float32)
    zi2 = jnp.zeros((32, 128), jnp.int32)
    t0 = _N - 64
    desc_tl = atom_description[t0:].astype(jnp.int32).T.reshape(-1)
    facc_tl = facc[t0:].T.reshape(-1)
    cr_tl = contRat[t0:].T.reshape(-1)
    hb_tl = atomHbonds[t0:].astype(jnp.int32).T.reshape(-1)
    crp_tl = jnp.transpose(contRatPol[t0:], (1, 2, 0)).reshape(-1)

    Pp, Hp = _sc_solv(desc_t, facc_t, cr_t, hb_t, crp_t,
                      props_f, frot_f, consts, zf, zf2, zi2,
                      desc_tl, facc_tl, cr_tl, hb_tl, crp_tl)

    P, H = pl.pallas_call(
        _tc_add,
        out_shape=(jax.ShapeDtypeStruct((500, 128), jnp.float32),
                   jax.ShapeDtypeStruct((500, 128), jnp.float32)),
    )(Pp.reshape(2, 501, 128), Hp.reshape(2, 501, 128))
    return P.reshape(2, 4, 2000, 4), H.reshape(2, 4, 2000, 4)
